# baseline (speedup 1.0000x reference)
import jax
import jax.numpy as jnp
from jax.experimental import pallas as pl
from jax.experimental.pallas import tpu as pltpu


_TOP_K = 8
_BLOCK_T = 1024


def _router_block(x_ref, mr_ref, mc_ref, w_ref,
                  logits_ref, probs_ref, wts_ref, idx_ref):
    x = x_ref[...]
    w = w_ref[...]
    mr = mr_ref[...]
    mc = mc_ref[...]
    e = w.shape[0]
    bt = x.shape[0]

    raw_n = jax.lax.dot_general(
        x, w, (((1,), (1,)), ((), ())), preferred_element_type=jnp.float32
    )
    logits_n = raw_n * (mc * mc)
    logits_ref[...] = logits_n
    mx_n = jnp.max(logits_n, axis=-1, keepdims=True)
    ex_n = jnp.exp(logits_n - mx_n)
    probs_ref[...] = (ex_n / jnp.sum(ex_n, axis=-1, keepdims=True)) * mc

    raw_t = jax.lax.dot_general(
        w, x, (((1,), (1,)), ((), ())), preferred_element_type=jnp.float32
    )
    logits_t = raw_t * (mr * mr)
    mx = jnp.max(logits_t, axis=0, keepdims=True)
    ex = jnp.exp(logits_t - mx)
    sm = ex / jnp.sum(ex, axis=0, keepdims=True)

    iota = jax.lax.broadcasted_iota(jnp.int32, (e, bt), 0)
    key = ((sm.view(jnp.int32) & jnp.int32(~63)) | (jnp.int32(e - 1) - iota))
    vals = []
    idxs = []
    for _ in range(_TOP_K):
        c = jnp.max(key, axis=0, keepdims=True)
        vals.append((c & jnp.int32(~63)).view(jnp.float32))
        idxs.append(jnp.int32(e - 1) - (c & jnp.int32(63)))
        key = jnp.where(key == c, jnp.int32(-1), key)
    wv = jnp.concatenate(vals, axis=0)
    iv = jnp.concatenate(idxs, axis=0)

    s = jnp.sum(wv, axis=0, keepdims=True)
    wv = wv / jnp.where(s > 0, s, jnp.ones_like(s))
    wts_ref[...] = wv * mr
    idx_ref[...] = jnp.where(mr != 0.0, iv, -1)


def kernel(x, x_mask, W):
    b, t, c = x.shape
    e = W.shape[0]
    n = b * t
    x2 = x.reshape(n, c)
    m_row = x_mask.reshape(1, n)
    m_col = x_mask.reshape(n, 1)

    grid = (n // _BLOCK_T,)
    logits, probs, wts_t, idx_t = pl.pallas_call(
        _router_block,
        grid=grid,
        in_specs=[
            pl.BlockSpec((_BLOCK_T, c), lambda i: (i, 0)),
            pl.BlockSpec((1, _BLOCK_T), lambda i: (0, i)),
            pl.BlockSpec((_BLOCK_T, 1), lambda i: (i, 0)),
            pl.BlockSpec((e, c), lambda i: (0, 0)),
        ],
        out_specs=[
            pl.BlockSpec((_BLOCK_T, e), lambda i: (i, 0)),
            pl.BlockSpec((_BLOCK_T, e), lambda i: (i, 0)),
            pl.BlockSpec((_TOP_K, _BLOCK_T), lambda i: (0, i)),
            pl.BlockSpec((_TOP_K, _BLOCK_T), lambda i: (0, i)),
        ],
        out_shape=[
            jax.ShapeDtypeStruct((n, e), jnp.float32),
            jax.ShapeDtypeStruct((n, e), jnp.float32),
            jax.ShapeDtypeStruct((_TOP_K, n), jnp.float32),
            jax.ShapeDtypeStruct((_TOP_K, n), jnp.int32),
        ],
        compiler_params=pltpu.CompilerParams(
            dimension_semantics=("arbitrary",),
        ),
    )(x2, m_row, m_col, W)

    return (
        wts_t.T.reshape(b, t, _TOP_K),
        idx_t.T.reshape(b, t, _TOP_K),
        logits.reshape(b, t, e),
        probs.reshape(b, t, e),
    )


# speedup vs baseline: 1.0751x; 1.0751x over previous
import jax
import jax.numpy as jnp
from jax.experimental import pallas as pl
from jax.experimental.pallas import tpu as pltpu


_TOP_K = 8
_BLOCK_T = 1024


def _mxu_t(a, ident):
    return jax.lax.dot_general(
        a, ident, (((0,), (0,)), ((), ())), preferred_element_type=jnp.float32
    )


def _router_block(x_ref, m_ref, w_ref, logits_ref, probs_ref, wts_ref, idx_ref):
    x = x_ref[...]
    w = w_ref[...]
    m = m_ref[...]
    e = w.shape[0]
    bt = x.shape[0]

    raw = jax.lax.dot_general(
        w, x, (((1,), (1,)), ((), ())), preferred_element_type=jnp.float32
    )
    logits = raw * (m * m)

    r1 = jax.lax.broadcasted_iota(jnp.int32, (e, e), 0)
    r2 = jax.lax.broadcasted_iota(jnp.int32, (e, e), 1)
    ident_e = jnp.where(r1 == r2, 1.0, 0.0).astype(jnp.float32)
    logits_ref[...] = _mxu_t(logits, ident_e)

    mx = jnp.max(logits, axis=0, keepdims=True)
    ex = jnp.exp(logits - mx)
    sm = ex / jnp.sum(ex, axis=0, keepdims=True)
    probs_ref[...] = _mxu_t(sm * m, ident_e)

    iota = jax.lax.broadcasted_iota(jnp.int32, (e, bt), 0)
    key = ((sm.view(jnp.int32) & jnp.int32(~63)) | (jnp.int32(e - 1) - iota))
    vals = []
    idxs = []
    for _ in range(_TOP_K):
        c = jnp.max(key, axis=0, keepdims=True)
        vals.append((c & jnp.int32(~63)).view(jnp.float32))
        idxs.append(jnp.int32(e - 1) - (c & jnp.int32(63)))
        key = jnp.where(key == c, jnp.int32(-1), key)
    wv = jnp.concatenate(vals, axis=0)
    iv = jnp.concatenate(idxs, axis=0)

    s = jnp.sum(wv, axis=0, keepdims=True)
    wv = wv / jnp.where(s > 0, s, jnp.ones_like(s))
    ident_k = ident_e[:_TOP_K, :_TOP_K]
    wts_ref[...] = _mxu_t(wv * m, ident_k)
    iv_masked = jnp.where(m != 0.0, iv, -1).astype(jnp.float32)
    idx_ref[...] = _mxu_t(iv_masked, ident_k).astype(jnp.int32)


def kernel(x, x_mask, W):
    b, t, c = x.shape
    e = W.shape[0]
    n = b * t
    x2 = x.reshape(n, c)
    m2 = x_mask.reshape(1, n)

    grid = (n // _BLOCK_T,)
    logits, probs, wts, idx = pl.pallas_call(
        _router_block,
        grid=grid,
        in_specs=[
            pl.BlockSpec((_BLOCK_T, c), lambda i: (i, 0)),
            pl.BlockSpec((1, _BLOCK_T), lambda i: (0, i)),
            pl.BlockSpec((e, c), lambda i: (0, 0)),
        ],
        out_specs=[
            pl.BlockSpec((_BLOCK_T, e), lambda i: (i, 0)),
            pl.BlockSpec((_BLOCK_T, e), lambda i: (i, 0)),
            pl.BlockSpec((_BLOCK_T, _TOP_K), lambda i: (i, 0)),
            pl.BlockSpec((_BLOCK_T, _TOP_K), lambda i: (i, 0)),
        ],
        out_shape=[
            jax.ShapeDtypeStruct((n, e), jnp.float32),
            jax.ShapeDtypeStruct((n, e), jnp.float32),
            jax.ShapeDtypeStruct((n, _TOP_K), jnp.float32),
            jax.ShapeDtypeStruct((n, _TOP_K), jnp.int32),
        ],
        compiler_params=pltpu.CompilerParams(
            dimension_semantics=("arbitrary",),
        ),
    )(x2, m2, W)

    return (
        wts.reshape(b, t, _TOP_K),
        idx.reshape(b, t, _TOP_K),
        logits.reshape(b, t, e),
        probs.reshape(b, t, e),
    )


# speedup vs baseline: 1.2470x; 1.1599x over previous
import jax
import jax.numpy as jnp
from jax.experimental import pallas as pl
from jax.experimental.pallas import tpu as pltpu


_TOP_K = 8
_BLOCK_T = 1024


def _router_block(xa_ref, xb_ref, m_ref, w_ref, logits_ref, probs_ref, wts_ref, idx_ref):
    xa = xa_ref[...]
    xb = xb_ref[...]
    w = w_ref[...]
    m = m_ref[...]
    e = w.shape[0]
    bt = 2 * xa.shape[0]

    raw = jnp.concatenate([
        jax.lax.dot_general(
            w, xa, (((1,), (1,)), ((), ())), preferred_element_type=jnp.float32
        ),
        jax.lax.dot_general(
            w, xb, (((1,), (1,)), ((), ())), preferred_element_type=jnp.float32
        ),
    ], axis=1)
    logits = raw * (m * m)
    logits_ref[...] = logits

    mx = jnp.max(logits, axis=0, keepdims=True)
    ex = jnp.exp(logits - mx)
    sm = ex / jnp.sum(ex, axis=0, keepdims=True)
    probs_ref[...] = sm * m

    iota = jax.lax.broadcasted_iota(jnp.int32, (e, bt), 0)
    key = ((sm.view(jnp.int32) & jnp.int32(~63)) | (jnp.int32(e - 1) - iota))
    vals = []
    idxs = []
    for _ in range(_TOP_K):
        c = jnp.max(key, axis=0, keepdims=True)
        vals.append((c & jnp.int32(~63)).view(jnp.float32))
        idxs.append(jnp.int32(e - 1) - (c & jnp.int32(63)))
        key = jnp.where(key == c, jnp.int32(-1), key)
    wv = jnp.concatenate(vals, axis=0)
    iv = jnp.concatenate(idxs, axis=0)

    s = jnp.sum(wv, axis=0, keepdims=True)
    wv = wv / jnp.where(s > 0, s, jnp.ones_like(s))
    wts_ref[...] = wv * m
    idx_ref[...] = jnp.where(m != 0.0, iv, -1)


def kernel(x, x_mask, W):
    b, t, c = x.shape
    e = W.shape[0]
    n = b * t
    x2 = x.reshape(n, c)
    m2 = x_mask.reshape(1, n)

    grid = (n // _BLOCK_T,)
    logits_t, probs_t, wts_t, idx_t = pl.pallas_call(
        _router_block,
        grid=grid,
        in_specs=[
            pl.BlockSpec((_BLOCK_T // 2, c), lambda i: (2 * i, 0)),
            pl.BlockSpec((_BLOCK_T // 2, c), lambda i: (2 * i + 1, 0)),
            pl.BlockSpec((1, _BLOCK_T), lambda i: (0, i)),
            pl.BlockSpec((e, c), lambda i: (0, 0)),
        ],
        out_specs=[
            pl.BlockSpec((e, _BLOCK_T), lambda i: (0, i)),
            pl.BlockSpec((e, _BLOCK_T), lambda i: (0, i)),
            pl.BlockSpec((_TOP_K, _BLOCK_T), lambda i: (0, i)),
            pl.BlockSpec((_TOP_K, _BLOCK_T), lambda i: (0, i)),
        ],
        out_shape=[
            jax.ShapeDtypeStruct((e, n), jnp.float32),
            jax.ShapeDtypeStruct((e, n), jnp.float32),
            jax.ShapeDtypeStruct((_TOP_K, n), jnp.float32),
            jax.ShapeDtypeStruct((_TOP_K, n), jnp.int32),
        ],
        compiler_params=pltpu.CompilerParams(
            dimension_semantics=("arbitrary",),
        ),
    )(x2, x2, m2, W)

    return (
        wts_t.T.reshape(b, t, _TOP_K),
        idx_t.T.reshape(b, t, _TOP_K),
        logits_t.T.reshape(b, t, e),
        probs_t.T.reshape(b, t, e),
    )


# speedup vs baseline: 1.2837x; 1.0294x over previous
import jax
import jax.numpy as jnp
from jax.experimental import pallas as pl
from jax.experimental.pallas import tpu as pltpu


_TOP_K = 8
_BLOCK_T = 1024


def _router_block(x_ref, m_ref, w_ref, logits_ref, probs_ref, wts_ref, idx_ref):
    x = x_ref[...]
    w = w_ref[...]
    m = m_ref[...]
    e = w.shape[0]
    bt = x.shape[0]

    raw = jax.lax.dot_general(
        w, x, (((1,), (1,)), ((), ())), preferred_element_type=jnp.float32
    )
    logits = raw * (m * m)
    logits_ref[...] = logits

    mx = jnp.max(logits, axis=0, keepdims=True)
    ex = jnp.exp(logits - mx)
    sm = ex / jnp.sum(ex, axis=0, keepdims=True)
    probs_ref[...] = sm * m

    iota = jax.lax.broadcasted_iota(jnp.int32, (e, bt), 0)
    key = ((sm.view(jnp.int32) & jnp.int32(~63)) | (jnp.int32(e - 1) - iota))
    vals = []
    idxs = []
    for _ in range(_TOP_K):
        c = jnp.max(key, axis=0, keepdims=True)
        vals.append((c & jnp.int32(~63)).view(jnp.float32))
        idxs.append(jnp.int32(e - 1) - (c & jnp.int32(63)))
        key = jnp.where(key == c, jnp.int32(-1), key)
    wv = jnp.concatenate(vals, axis=0)
    iv = jnp.concatenate(idxs, axis=0)

    s = jnp.sum(wv, axis=0, keepdims=True)
    wv = wv / jnp.where(s > 0, s, jnp.ones_like(s))
    wts_ref[...] = wv * m
    idx_ref[...] = jnp.where(m != 0.0, iv, -1)


def kernel(x, x_mask, W):
    b, t, c = x.shape
    e = W.shape[0]
    n = b * t
    x2 = x.reshape(n, c)
    m2 = x_mask.reshape(1, n)

    grid = (n // _BLOCK_T,)
    logits_t, probs_t, wts_t, idx_t = pl.pallas_call(
        _router_block,
        grid=grid,
        in_specs=[
            pl.BlockSpec((_BLOCK_T, c), lambda i: (i, 0)),
            pl.BlockSpec((1, _BLOCK_T), lambda i: (0, i)),
            pl.BlockSpec((e, c), lambda i: (0, 0)),
        ],
        out_specs=[
            pl.BlockSpec((e, _BLOCK_T), lambda i: (0, i)),
            pl.BlockSpec((e, _BLOCK_T), lambda i: (0, i)),
            pl.BlockSpec((_TOP_K, _BLOCK_T), lambda i: (0, i)),
            pl.BlockSpec((_TOP_K, _BLOCK_T), lambda i: (0, i)),
        ],
        out_shape=[
            jax.ShapeDtypeStruct((e, n), jnp.float32),
            jax.ShapeDtypeStruct((e, n), jnp.float32),
            jax.ShapeDtypeStruct((_TOP_K, n), jnp.float32),
            jax.ShapeDtypeStruct((_TOP_K, n), jnp.int32),
        ],
        compiler_params=pltpu.CompilerParams(
            dimension_semantics=("arbitrary",),
        ),
    )(x2, m2, W)

    return (
        wts_t.T.reshape(b, t, _TOP_K),
        idx_t.T.reshape(b, t, _TOP_K),
        logits_t.T.reshape(b, t, e),
        probs_t.T.reshape(b, t, e),
    )
